# Initial kernel scaffold; baseline (speedup 1.0000x reference)
import jax
import jax.numpy as jnp
from jax import lax
from jax.experimental import pallas as pl
from jax.experimental.pallas import tpu as pltpu

_CUTOFF2 = 100.0
_K = 32
_QT = 16
_WT = 512
_N = 10000
_NB = (_N + _WT - 1) // _WT + 1
_NPADQ = _N + _QT
_BIG = jnp.int32(1 << 30)


def _body(starts_ref, posq_ref, posblk_ref, outi_ref, outw_ref, buf_ref):
    b = pl.program_id(0)
    s = starts_ref[b]
    e = starts_ref[b + 1]
    q0base = (s // _QT) * _QT
    nq = (e - q0base + _QT - 1) // _QT
    wb0 = s // _WT
    nb = jnp.maximum(0, (e - 1) // _WT - wb0 + 1)
    lane = lax.broadcasted_iota(jnp.int32, (_QT, _WT), 1)
    kiota = lax.broadcasted_iota(jnp.int32, (_QT, _K), 1)

    def tile_body(qt, _):
        q0 = q0base + qt * _QT
        qp = posq_ref[pl.ds(q0, _QT), :]
        qx = qp[:, 0:1]
        qy = qp[:, 1:2]
        qz = qp[:, 2:3]
        qi = q0 + lax.broadcasted_iota(jnp.int32, (_QT, 1), 0)

        def fill(lb, _):
            wb = wb0 + lb
            p = posblk_ref[wb]
            dx = qx - p[0:1, :]
            dy = qy - p[1:2, :]
            dz = qz - p[2:3, :]
            d2 = (dx * dx + dy * dy) + dz * dz
            colg = wb * _WT + lane
            valid = (colg >= s) & (colg < e) & (colg != qi) & (d2 <= _CUTOFF2)
            buf_ref[lb] = jnp.where(valid, d2, jnp.inf)
            return 0

        lax.fori_loop(0, nb, fill, 0, unroll=False)

        m_prev = jnp.full((_QT, 1), -jnp.inf, jnp.float32)
        i_prev = jnp.full((_QT, 1), -1, jnp.int32)
        acc_i = jnp.broadcast_to(qi, (_QT, _K))
        acc_w = jnp.zeros((_QT, _K), jnp.float32)
        for it in range(_K):
            def scan(lb, carry):
                bm, bi = carry
                blk = buf_ref[lb]
                colg = (wb0 + lb) * _WT + lane
                succ = (blk > m_prev) | ((blk == m_prev) & (colg > i_prev))
                cand = jnp.where(succ, blk, jnp.inf)
                cm = jnp.min(cand, axis=1, keepdims=True)
                ci = jnp.min(jnp.where(cand == cm, colg, _BIG), axis=1,
                             keepdims=True)
                better = (cm < bm) | ((cm == bm) & (ci < bi))
                return (jnp.where(better, cm, bm), jnp.where(better, ci, bi))

            m, i = lax.fori_loop(
                0, nb, scan,
                (jnp.full((_QT, 1), jnp.inf, jnp.float32),
                 jnp.full((_QT, 1), _BIG, jnp.int32)))
            valid = m < jnp.inf
            sel_i = jnp.where(valid, i, qi)
            sel_w = jnp.where(valid, jnp.sqrt(m), 0.0)
            hit = kiota == it
            acc_i = jnp.where(hit, jnp.broadcast_to(sel_i, (_QT, _K)), acc_i)
            acc_w = jnp.where(hit, jnp.broadcast_to(sel_w, (_QT, _K)), acc_w)
            m_prev, i_prev = m, i

        rowok = (qi >= s) & (qi < e)
        cur_i = outi_ref[pl.ds(q0, _QT), :]
        cur_w = outw_ref[pl.ds(q0, _QT), :]
        outi_ref[pl.ds(q0, _QT), :] = jnp.where(rowok, acc_i, cur_i)
        outw_ref[pl.ds(q0, _QT), :] = jnp.where(rowok, acc_w, cur_w)
        return 0

    lax.fori_loop(0, nq, tile_body, 0, unroll=False)


def _radius_graph(pos, batch):
    n = pos.shape[0]
    starts = jnp.searchsorted(
        batch, jnp.arange(17, dtype=jnp.int32), side="left").astype(jnp.int32)
    posq = jnp.pad(pos, ((0, _NPADQ - n), (0, 0)))
    pos_t = jnp.pad(pos.T, ((0, 0), (0, _NB * _WT - n)))
    posblk = pos_t.reshape(3, _NB, _WT).transpose(1, 0, 2)

    grid_spec = pltpu.PrefetchScalarGridSpec(
        num_scalar_prefetch=1,
        grid=(16,),
        in_specs=[
            pl.BlockSpec(posq.shape, lambda b, starts: (0, 0)),
            pl.BlockSpec(posblk.shape, lambda b, starts: (0, 0, 0)),
        ],
        out_specs=[
            pl.BlockSpec((_NPADQ, _K), lambda b, starts: (0, 0)),
            pl.BlockSpec((_NPADQ, _K), lambda b, starts: (0, 0)),
        ],
        scratch_shapes=[pltpu.VMEM((_NB, _QT, _WT), jnp.float32)],
    )
    out_i, out_w = pl.pallas_call(
        _body,
        grid_spec=grid_spec,
        out_shape=[
            jax.ShapeDtypeStruct((_NPADQ, _K), jnp.int32),
            jax.ShapeDtypeStruct((_NPADQ, _K), jnp.float32),
        ],
    )(starts, posq, posblk)

    row = out_i[:n].reshape(-1)
    col = jnp.broadcast_to(
        jnp.arange(n, dtype=jnp.int32)[:, None], (n, _K)).reshape(-1)
    edge_index = jnp.stack([row, col], axis=0)
    edge_weight = out_w[:n].reshape(-1)
    return edge_index, edge_weight


def kernel(pos, batch):
    return _radius_graph(pos, batch)


# speedup vs baseline: 1.8764x; 1.8764x over previous
import jax
import jax.numpy as jnp
from jax import lax
from jax.experimental import pallas as pl
from jax.experimental.pallas import tpu as pltpu

_CUTOFF2 = 100.0
_K = 32
_QT = 16
_WT = 512
_N = 10000
_NB = (_N + _WT - 1) // _WT + 1
_NPADQ = _N + _QT
_BIG = 1 << 30


def _body(starts_ref, posq_ref, posblk_ref, outi_ref, outw_ref, buf_ref):
    b = pl.program_id(0)
    s = starts_ref[b]
    e = starts_ref[b + 1]
    q0base = (s // _QT) * _QT
    nq = (e - q0base + _QT - 1) // _QT
    wb0 = s // _WT
    nb = jnp.maximum(0, (e - 1) // _WT - wb0 + 1)
    lane = lax.broadcasted_iota(jnp.int32, (_QT, _WT), 1)
    kiota = lax.broadcasted_iota(jnp.int32, (_QT, _K), 1)

    def tile_body(qt, _):
        q0 = q0base + qt * _QT
        qp = posq_ref[pl.ds(q0, _QT), :]
        qx = qp[:, 0:1]
        qy = qp[:, 1:2]
        qz = qp[:, 2:3]
        qi = q0 + lax.broadcasted_iota(jnp.int32, (_QT, 1), 0)

        def fill(lb, _):
            wb = wb0 + lb
            p = posblk_ref[wb]
            dx = qx - p[0:1, :]
            dy = qy - p[1:2, :]
            dz = qz - p[2:3, :]
            d2 = (dx * dx + dy * dy) + dz * dz
            colg = wb * _WT + lane
            valid = (colg >= s) & (colg < e) & (colg != qi) & (d2 <= _CUTOFF2)
            buf_ref[lb] = jnp.where(valid, d2, jnp.inf)
            return 0

        lax.fori_loop(0, nb, fill, 0, unroll=False)

        m_prev = jnp.full((_QT, 1), -jnp.inf, jnp.float32)
        i_prev = jnp.full((_QT, 1), -1, jnp.int32)
        acc_i = jnp.broadcast_to(qi, (_QT, _K))
        acc_w = jnp.zeros((_QT, _K), jnp.float32)
        for it in range(_K):
            def scan(lb, carry):
                bm, bi = carry
                blk = buf_ref[lb]
                colg = (wb0 + lb) * _WT + lane
                succ = (blk > m_prev) | ((blk == m_prev) & (colg > i_prev))
                cand = jnp.where(succ, blk, jnp.inf)
                cm = jnp.min(cand, axis=1, keepdims=True)
                ci = jnp.min(jnp.where(cand == cm, colg, _BIG), axis=1,
                             keepdims=True)
                better = (cm < bm) | ((cm == bm) & (ci < bi))
                return (jnp.where(better, cm, bm), jnp.where(better, ci, bi))

            m, i = lax.fori_loop(
                0, nb, scan,
                (jnp.full((_QT, 1), jnp.inf, jnp.float32),
                 jnp.full((_QT, 1), _BIG, jnp.int32)))
            valid = m < jnp.inf
            sel_i = jnp.where(valid, i, qi)
            sel_w = jnp.where(valid, jnp.sqrt(m), 0.0)
            hit = kiota == it
            acc_i = jnp.where(hit, jnp.broadcast_to(sel_i, (_QT, _K)), acc_i)
            acc_w = jnp.where(hit, jnp.broadcast_to(sel_w, (_QT, _K)), acc_w)
            m_prev, i_prev = m, i

        rowok = (qi >= s) & (qi < e)
        cur_i = outi_ref[pl.ds(q0, _QT), :]
        cur_w = outw_ref[pl.ds(q0, _QT), :]
        outi_ref[pl.ds(q0, _QT), :] = jnp.where(rowok, acc_i, cur_i)
        outw_ref[pl.ds(q0, _QT), :] = jnp.where(rowok, acc_w, cur_w)
        return 0

    lax.fori_loop(0, nq, tile_body, 0, unroll=False)


def _radius_graph(pos, batch):
    n = pos.shape[0]
    starts = jnp.searchsorted(
        batch, jnp.arange(17, dtype=jnp.int32), side="left").astype(jnp.int32)
    posq = jnp.pad(pos, ((0, _NPADQ - n), (0, 0)))
    pos_t = jnp.pad(pos.T, ((0, 0), (0, _NB * _WT - n)))
    posblk = pos_t.reshape(3, _NB, _WT).transpose(1, 0, 2)

    grid_spec = pltpu.PrefetchScalarGridSpec(
        num_scalar_prefetch=1,
        grid=(16,),
        in_specs=[
            pl.BlockSpec(posq.shape, lambda b, starts: (0, 0)),
            pl.BlockSpec(posblk.shape, lambda b, starts: (0, 0, 0)),
        ],
        out_specs=[
            pl.BlockSpec((_NPADQ, _K), lambda b, starts: (0, 0)),
            pl.BlockSpec((_NPADQ, _K), lambda b, starts: (0, 0)),
        ],
        scratch_shapes=[pltpu.VMEM((_NB, _QT, _WT), jnp.float32)],
    )
    out_i, out_w = pl.pallas_call(
        _body,
        grid_spec=grid_spec,
        out_shape=[
            jax.ShapeDtypeStruct((_NPADQ, _K), jnp.int32),
            jax.ShapeDtypeStruct((_NPADQ, _K), jnp.float32),
        ],
    )(starts, posq, posblk)

    row = out_i[:n].reshape(-1)
    col = jnp.broadcast_to(
        jnp.arange(n, dtype=jnp.int32)[:, None], (n, _K)).reshape(-1)
    edge_index = jnp.stack([row, col], axis=0)
    edge_weight = out_w[:n].reshape(-1)
    return edge_index, edge_weight


def kernel(pos, batch):
    return _radius_graph(pos, batch)


# speedup vs baseline: 6.0529x; 3.2259x over previous
import jax
import jax.numpy as jnp
from jax import lax
from jax.experimental import pallas as pl
from jax.experimental.pallas import tpu as pltpu

_CUTOFF2 = 100.0
_K = 32
_QT = 64
_WT = 512
_N = 10000
_NB = (_N + _WT - 1) // _WT + 1
_NPADQ = _N + _QT
_BIG = 1 << 30


def _body(starts_ref, posq_ref, posblk_ref, outi_ref, outw_ref, buf_ref):
    b = pl.program_id(0)
    s = starts_ref[b]
    e = starts_ref[b + 1]
    q0base = (s // _QT) * _QT
    nq = (e - q0base + _QT - 1) // _QT
    wb0 = s // _WT
    nb = jnp.maximum(0, (e - 1) // _WT - wb0 + 1)
    lane = lax.broadcasted_iota(jnp.int32, (_QT, _WT), 1)
    kiota = lax.broadcasted_iota(jnp.int32, (_QT, _K), 1)

    def tile_body(qt, _):
        q0 = q0base + qt * _QT
        qp = posq_ref[pl.ds(q0, _QT), :]
        qx = qp[:, 0:1]
        qy = qp[:, 1:2]
        qz = qp[:, 2:3]
        qi = q0 + lax.broadcasted_iota(jnp.int32, (_QT, 1), 0)

        def fill(lb, _):
            wb = wb0 + lb
            p = posblk_ref[wb]
            dx = qx - p[0:1, :]
            dy = qy - p[1:2, :]
            dz = qz - p[2:3, :]
            d2 = (dx * dx + dy * dy) + dz * dz
            colg = wb * _WT + lane
            valid = (colg >= s) & (colg < e) & (colg != qi) & (d2 <= _CUTOFF2)
            buf_ref[lb] = jnp.where(valid, d2, jnp.inf)
            return 0

        lax.fori_loop(0, nb, fill, 0, unroll=False)

        m_prev = jnp.full((_QT, 1), -jnp.inf, jnp.float32)
        i_prev = jnp.full((_QT, 1), -1, jnp.int32)
        acc_i = jnp.broadcast_to(qi, (_QT, _K))
        acc_w = jnp.zeros((_QT, _K), jnp.float32)
        for it in range(_K):
            def scan(lb, carry):
                bm, bi = carry
                blk = buf_ref[lb]
                colg = (wb0 + lb) * _WT + lane
                succ = (blk > m_prev) | ((blk == m_prev) & (colg > i_prev))
                cand = jnp.where(succ, blk, jnp.inf)
                cm = jnp.min(cand, axis=1, keepdims=True)
                ci = jnp.min(jnp.where(cand == cm, colg, _BIG), axis=1,
                             keepdims=True)
                better = (cm < bm) | ((cm == bm) & (ci < bi))
                return (jnp.where(better, cm, bm), jnp.where(better, ci, bi))

            m, i = lax.fori_loop(
                0, nb, scan,
                (jnp.full((_QT, 1), jnp.inf, jnp.float32),
                 jnp.full((_QT, 1), _BIG, jnp.int32)))
            valid = m < jnp.inf
            sel_i = jnp.where(valid, i, qi)
            sel_w = jnp.where(valid, jnp.sqrt(m), 0.0)
            hit = kiota == it
            acc_i = jnp.where(hit, jnp.broadcast_to(sel_i, (_QT, _K)), acc_i)
            acc_w = jnp.where(hit, jnp.broadcast_to(sel_w, (_QT, _K)), acc_w)
            m_prev, i_prev = m, i

        rowok = (qi >= s) & (qi < e)
        cur_i = outi_ref[pl.ds(q0, _QT), :]
        cur_w = outw_ref[pl.ds(q0, _QT), :]
        outi_ref[pl.ds(q0, _QT), :] = jnp.where(rowok, acc_i, cur_i)
        outw_ref[pl.ds(q0, _QT), :] = jnp.where(rowok, acc_w, cur_w)
        return 0

    lax.fori_loop(0, nq, tile_body, 0, unroll=False)


def _radius_graph(pos, batch):
    n = pos.shape[0]
    starts = jnp.searchsorted(
        batch, jnp.arange(17, dtype=jnp.int32), side="left").astype(jnp.int32)
    posq = jnp.pad(pos, ((0, _NPADQ - n), (0, 0)))
    pos_t = jnp.pad(pos.T, ((0, 0), (0, _NB * _WT - n)))
    posblk = pos_t.reshape(3, _NB, _WT).transpose(1, 0, 2)

    grid_spec = pltpu.PrefetchScalarGridSpec(
        num_scalar_prefetch=1,
        grid=(16,),
        in_specs=[
            pl.BlockSpec(posq.shape, lambda b, starts: (0, 0)),
            pl.BlockSpec(posblk.shape, lambda b, starts: (0, 0, 0)),
        ],
        out_specs=[
            pl.BlockSpec((_NPADQ, _K), lambda b, starts: (0, 0)),
            pl.BlockSpec((_NPADQ, _K), lambda b, starts: (0, 0)),
        ],
        scratch_shapes=[pltpu.VMEM((_NB, _QT, _WT), jnp.float32)],
    )
    out_i, out_w = pl.pallas_call(
        _body,
        grid_spec=grid_spec,
        out_shape=[
            jax.ShapeDtypeStruct((_NPADQ, _K), jnp.int32),
            jax.ShapeDtypeStruct((_NPADQ, _K), jnp.float32),
        ],
    )(starts, posq, posblk)

    row = out_i[:n].reshape(-1)
    col = jnp.broadcast_to(
        jnp.arange(n, dtype=jnp.int32)[:, None], (n, _K)).reshape(-1)
    edge_index = jnp.stack([row, col], axis=0)
    edge_weight = out_w[:n].reshape(-1)
    return edge_index, edge_weight


def kernel(pos, batch):
    return _radius_graph(pos, batch)


# speedup vs baseline: 9.4098x; 1.5546x over previous
import jax
import jax.numpy as jnp
from jax import lax
from jax.experimental import pallas as pl
from jax.experimental.pallas import tpu as pltpu

_CUTOFF2 = 100.0
_K = 32
_QT = 128
_WT = 512
_N = 10000
_NB = (_N + _WT - 1) // _WT + 1
_NPADQ = _N + _QT
_BIG = 1 << 30


def _body(starts_ref, posq_ref, posblk_ref, outi_ref, outw_ref, buf_ref):
    b = pl.program_id(0)
    s = starts_ref[b]
    e = starts_ref[b + 1]
    q0base = (s // _QT) * _QT
    nq = (e - q0base + _QT - 1) // _QT
    wb0 = s // _WT
    nb = jnp.maximum(0, (e - 1) // _WT - wb0 + 1)
    lane = lax.broadcasted_iota(jnp.int32, (_QT, _WT), 1)
    kiota = lax.broadcasted_iota(jnp.int32, (_QT, _K), 1)

    def tile_body(qt, _):
        q0 = q0base + qt * _QT
        qp = posq_ref[pl.ds(q0, _QT), :]
        qx = qp[:, 0:1]
        qy = qp[:, 1:2]
        qz = qp[:, 2:3]
        qi = q0 + lax.broadcasted_iota(jnp.int32, (_QT, 1), 0)

        def fill(lb, _):
            wb = wb0 + lb
            p = posblk_ref[wb]
            dx = qx - p[0:1, :]
            dy = qy - p[1:2, :]
            dz = qz - p[2:3, :]
            d2 = (dx * dx + dy * dy) + dz * dz
            colg = wb * _WT + lane
            valid = (colg >= s) & (colg < e) & (colg != qi) & (d2 <= _CUTOFF2)
            buf_ref[lb] = jnp.where(valid, d2, jnp.inf)
            return 0

        lax.fori_loop(0, nb, fill, 0, unroll=False)

        m_prev = jnp.full((_QT, 1), -jnp.inf, jnp.float32)
        i_prev = jnp.full((_QT, 1), -1, jnp.int32)
        acc_i = jnp.broadcast_to(qi, (_QT, _K))
        acc_w = jnp.zeros((_QT, _K), jnp.float32)
        for it in range(_K):
            def scan(lb, carry):
                bm, bi = carry
                blk = buf_ref[lb]
                colg = (wb0 + lb) * _WT + lane
                succ = (blk > m_prev) | ((blk == m_prev) & (colg > i_prev))
                cand = jnp.where(succ, blk, jnp.inf)
                cm = jnp.min(cand, axis=1, keepdims=True)
                ci = jnp.min(jnp.where(cand == cm, colg, _BIG), axis=1,
                             keepdims=True)
                better = (cm < bm) | ((cm == bm) & (ci < bi))
                return (jnp.where(better, cm, bm), jnp.where(better, ci, bi))

            m, i = lax.fori_loop(
                0, nb, scan,
                (jnp.full((_QT, 1), jnp.inf, jnp.float32),
                 jnp.full((_QT, 1), _BIG, jnp.int32)))
            valid = m < jnp.inf
            sel_i = jnp.where(valid, i, qi)
            sel_w = jnp.where(valid, jnp.sqrt(m), 0.0)
            hit = kiota == it
            acc_i = jnp.where(hit, jnp.broadcast_to(sel_i, (_QT, _K)), acc_i)
            acc_w = jnp.where(hit, jnp.broadcast_to(sel_w, (_QT, _K)), acc_w)
            m_prev, i_prev = m, i

        rowok = (qi >= s) & (qi < e)
        cur_i = outi_ref[pl.ds(q0, _QT), :]
        cur_w = outw_ref[pl.ds(q0, _QT), :]
        outi_ref[pl.ds(q0, _QT), :] = jnp.where(rowok, acc_i, cur_i)
        outw_ref[pl.ds(q0, _QT), :] = jnp.where(rowok, acc_w, cur_w)
        return 0

    lax.fori_loop(0, nq, tile_body, 0, unroll=False)


def _radius_graph(pos, batch):
    n = pos.shape[0]
    starts = jnp.searchsorted(
        batch, jnp.arange(17, dtype=jnp.int32), side="left").astype(jnp.int32)
    posq = jnp.pad(pos, ((0, _NPADQ - n), (0, 0)))
    pos_t = jnp.pad(pos.T, ((0, 0), (0, _NB * _WT - n)))
    posblk = pos_t.reshape(3, _NB, _WT).transpose(1, 0, 2)

    grid_spec = pltpu.PrefetchScalarGridSpec(
        num_scalar_prefetch=1,
        grid=(16,),
        in_specs=[
            pl.BlockSpec(posq.shape, lambda b, starts: (0, 0)),
            pl.BlockSpec(posblk.shape, lambda b, starts: (0, 0, 0)),
        ],
        out_specs=[
            pl.BlockSpec((_NPADQ, _K), lambda b, starts: (0, 0)),
            pl.BlockSpec((_NPADQ, _K), lambda b, starts: (0, 0)),
        ],
        scratch_shapes=[pltpu.VMEM((_NB, _QT, _WT), jnp.float32)],
    )
    out_i, out_w = pl.pallas_call(
        _body,
        grid_spec=grid_spec,
        out_shape=[
            jax.ShapeDtypeStruct((_NPADQ, _K), jnp.int32),
            jax.ShapeDtypeStruct((_NPADQ, _K), jnp.float32),
        ],
    )(starts, posq, posblk)

    row = out_i[:n].reshape(-1)
    col = jnp.broadcast_to(
        jnp.arange(n, dtype=jnp.int32)[:, None], (n, _K)).reshape(-1)
    edge_index = jnp.stack([row, col], axis=0)
    edge_weight = out_w[:n].reshape(-1)
    return edge_index, edge_weight


def kernel(pos, batch):
    return _radius_graph(pos, batch)


# speedup vs baseline: 11.6947x; 1.2428x over previous
import jax
import jax.numpy as jnp
from jax import lax
from jax.experimental import pallas as pl
from jax.experimental.pallas import tpu as pltpu

_CUTOFF2 = 100.0
_K = 32
_QT = 256
_WT = 512
_N = 10000
_NB = (_N + _WT - 1) // _WT + 1
_NPADQ = _N + _QT
_BIG = 1 << 30


def _body(starts_ref, posq_ref, posblk_ref, outi_ref, outw_ref, buf_ref):
    b = pl.program_id(0)
    s = starts_ref[b]
    e = starts_ref[b + 1]
    q0base = (s // _QT) * _QT
    nq = (e - q0base + _QT - 1) // _QT
    wb0 = s // _WT
    nb = jnp.maximum(0, (e - 1) // _WT - wb0 + 1)
    lane = lax.broadcasted_iota(jnp.int32, (_QT, _WT), 1)
    kiota = lax.broadcasted_iota(jnp.int32, (_QT, _K), 1)

    def tile_body(qt, _):
        q0 = q0base + qt * _QT
        qp = posq_ref[pl.ds(q0, _QT), :]
        qx = qp[:, 0:1]
        qy = qp[:, 1:2]
        qz = qp[:, 2:3]
        qi = q0 + lax.broadcasted_iota(jnp.int32, (_QT, 1), 0)

        def fill(lb, _):
            wb = wb0 + lb
            p = posblk_ref[wb]
            dx = qx - p[0:1, :]
            dy = qy - p[1:2, :]
            dz = qz - p[2:3, :]
            d2 = (dx * dx + dy * dy) + dz * dz
            colg = wb * _WT + lane
            valid = (colg >= s) & (colg < e) & (colg != qi) & (d2 <= _CUTOFF2)
            buf_ref[lb] = jnp.where(valid, d2, jnp.inf)
            return 0

        lax.fori_loop(0, nb, fill, 0, unroll=False)

        m_prev = jnp.full((_QT, 1), -jnp.inf, jnp.float32)
        i_prev = jnp.full((_QT, 1), -1, jnp.int32)
        acc_i = jnp.broadcast_to(qi, (_QT, _K))
        acc_w = jnp.zeros((_QT, _K), jnp.float32)
        for it in range(_K):
            def scan(lb, carry):
                bm, bi = carry
                blk = buf_ref[lb]
                colg = (wb0 + lb) * _WT + lane
                succ = (blk > m_prev) | ((blk == m_prev) & (colg > i_prev))
                cand = jnp.where(succ, blk, jnp.inf)
                cm = jnp.min(cand, axis=1, keepdims=True)
                ci = jnp.min(jnp.where(cand == cm, colg, _BIG), axis=1,
                             keepdims=True)
                better = (cm < bm) | ((cm == bm) & (ci < bi))
                return (jnp.where(better, cm, bm), jnp.where(better, ci, bi))

            m, i = lax.fori_loop(
                0, nb, scan,
                (jnp.full((_QT, 1), jnp.inf, jnp.float32),
                 jnp.full((_QT, 1), _BIG, jnp.int32)))
            valid = m < jnp.inf
            sel_i = jnp.where(valid, i, qi)
            sel_w = jnp.where(valid, jnp.sqrt(m), 0.0)
            hit = kiota == it
            acc_i = jnp.where(hit, jnp.broadcast_to(sel_i, (_QT, _K)), acc_i)
            acc_w = jnp.where(hit, jnp.broadcast_to(sel_w, (_QT, _K)), acc_w)
            m_prev, i_prev = m, i

        rowok = (qi >= s) & (qi < e)
        cur_i = outi_ref[pl.ds(q0, _QT), :]
        cur_w = outw_ref[pl.ds(q0, _QT), :]
        outi_ref[pl.ds(q0, _QT), :] = jnp.where(rowok, acc_i, cur_i)
        outw_ref[pl.ds(q0, _QT), :] = jnp.where(rowok, acc_w, cur_w)
        return 0

    lax.fori_loop(0, nq, tile_body, 0, unroll=False)


def _radius_graph(pos, batch):
    n = pos.shape[0]
    starts = jnp.searchsorted(
        batch, jnp.arange(17, dtype=jnp.int32), side="left").astype(jnp.int32)
    posq = jnp.pad(pos, ((0, _NPADQ - n), (0, 0)))
    pos_t = jnp.pad(pos.T, ((0, 0), (0, _NB * _WT - n)))
    posblk = pos_t.reshape(3, _NB, _WT).transpose(1, 0, 2)

    grid_spec = pltpu.PrefetchScalarGridSpec(
        num_scalar_prefetch=1,
        grid=(16,),
        in_specs=[
            pl.BlockSpec(posq.shape, lambda b, starts: (0, 0)),
            pl.BlockSpec(posblk.shape, lambda b, starts: (0, 0, 0)),
        ],
        out_specs=[
            pl.BlockSpec((_NPADQ, _K), lambda b, starts: (0, 0)),
            pl.BlockSpec((_NPADQ, _K), lambda b, starts: (0, 0)),
        ],
        scratch_shapes=[pltpu.VMEM((_NB, _QT, _WT), jnp.float32)],
    )
    out_i, out_w = pl.pallas_call(
        _body,
        grid_spec=grid_spec,
        out_shape=[
            jax.ShapeDtypeStruct((_NPADQ, _K), jnp.int32),
            jax.ShapeDtypeStruct((_NPADQ, _K), jnp.float32),
        ],
    )(starts, posq, posblk)

    row = out_i[:n].reshape(-1)
    col = jnp.broadcast_to(
        jnp.arange(n, dtype=jnp.int32)[:, None], (n, _K)).reshape(-1)
    edge_index = jnp.stack([row, col], axis=0)
    edge_weight = out_w[:n].reshape(-1)
    return edge_index, edge_weight


def kernel(pos, batch):
    return _radius_graph(pos, batch)


# speedup vs baseline: 36.1955x; 3.0950x over previous
import functools

import jax
import jax.numpy as jnp
from jax import lax
from jax.experimental import pallas as pl
from jax.experimental.pallas import tpu as pltpu
from jax.experimental.pallas import tpu_sc as plsc

_CUTOFF2 = 100.0
_K = 32
_K2 = 48
_N = 10000
_L = 16
_NW = 32
_NPAD = 10240
_QPW = _NPAD // _NW
_BIG = 1 << 30

_INF = float("inf")


def _sc_body(x_hbm, y_hbm, z_hbm, ss_hbm, se_hbm, okey_hbm, oval_hbm,
             xv, yv, zv, ssv, sev, okeys, ovals):
    wid = lax.axis_index("s") * 2 + lax.axis_index("c")
    base = wid * _QPW
    pltpu.sync_copy(x_hbm, xv)
    pltpu.sync_copy(y_hbm, yv)
    pltpu.sync_copy(z_hbm, zv)
    pltpu.sync_copy(ss_hbm.at[pl.ds(base, _QPW)], ssv)
    pltpu.sync_copy(se_hbm.at[pl.ds(base, _QPW)], sev)

    lane = lax.iota(jnp.int32, _L)

    def per_group(g, _):
        g0 = g * _L
        qxc = xv[pl.ds(base + g0, _L)]
        qyc = yv[pl.ds(base + g0, _L)]
        qzc = zv[pl.ds(base + g0, _L)]
        ssc = ssv[pl.ds(g0, _L)]
        sec = sev[pl.ds(g0, _L)]
        for i in range(_L):
            qloc = g0 + i
            q = base + qloc
            qx = qxc[i]
            qy = qyc[i]
            qz = qzc[i]
            s = ssc[i]
            e = sec[i]

            t0k = jnp.full((_L,), _INF, jnp.float32)
            t1k = jnp.full((_L,), _INF, jnp.float32)
            t2k = jnp.full((_L,), _INF, jnp.float32)
            t0v = jnp.zeros((_L,), jnp.int32)
            t1v = jnp.zeros((_L,), jnp.int32)
            t2v = jnp.zeros((_L,), jnp.int32)

            def chunk_body(c, carry):
                t0k, t0v, t1k, t1v, t2k, t2v = carry
                j0 = c * _L
                gidx = j0 + lane
                dx = xv[pl.ds(j0, _L)] - qx
                dy = yv[pl.ds(j0, _L)] - qy
                dz = zv[pl.ds(j0, _L)] - qz
                d2 = (dx * dx + dy * dy) + dz * dz
                valid = ((gidx >= s) & (gidx < e) & (gidx != q)
                         & (d2 <= _CUTOFF2))
                dkey = jnp.where(valid, d2, _INF)

                ck, cv = plsc.sort_key_val(dkey, gidx)
                rk = lax.rev(ck, (0,))
                rv = lax.rev(cv, (0,))
                sel = t2k <= rk
                lo2k = jnp.where(sel, t2k, rk)
                lo2v = jnp.where(sel, t2v, rv)
                mk, mv = plsc.sort_key_val(lo2k, lo2v)

                rmk = lax.rev(mk, (0,))
                rmv = lax.rev(mv, (0,))
                sel1 = t1k <= rmk
                lo1k = jnp.where(sel1, t1k, rmk)
                lo1v = jnp.where(sel1, t1v, rmv)
                hi1k = jnp.where(sel1, rmk, t1k)
                hi1v = jnp.where(sel1, rmv, t1v)
                nt2k, nt2v = plsc.sort_key_val(hi1k, hi1v)
                l1k, l1v = plsc.sort_key_val(lo1k, lo1v)

                rlk = lax.rev(l1k, (0,))
                rlv = lax.rev(l1v, (0,))
                sel0 = t0k <= rlk
                lo0k = jnp.where(sel0, t0k, rlk)
                lo0v = jnp.where(sel0, t0v, rlv)
                hi0k = jnp.where(sel0, rlk, t0k)
                hi0v = jnp.where(sel0, rlv, t0v)
                nt0k, nt0v = plsc.sort_key_val(lo0k, lo0v)
                nt1k, nt1v = plsc.sort_key_val(hi0k, hi0v)
                return nt0k, nt0v, nt1k, nt1v, nt2k, nt2v

            c_lo = s // _L
            c_hi = (e - 1) // _L + 1
            t0k, t0v, t1k, t1v, t2k, t2v = lax.fori_loop(
                c_lo, c_hi, chunk_body,
                (t0k, t0v, t1k, t1v, t2k, t2v))

            off = qloc * _K2
            okeys[pl.ds(off, _L)] = t0k
            okeys[pl.ds(off + _L, _L)] = t1k
            okeys[pl.ds(off + 2 * _L, _L)] = t2k
            ovals[pl.ds(off, _L)] = t0v
            ovals[pl.ds(off + _L, _L)] = t1v
            ovals[pl.ds(off + 2 * _L, _L)] = t2v
        return 0

    lax.fori_loop(0, _QPW // _L, per_group, 0)
    pltpu.sync_copy(okeys, okey_hbm.at[pl.ds(base * _K2, _QPW * _K2)])
    pltpu.sync_copy(ovals, oval_hbm.at[pl.ds(base * _K2, _QPW * _K2)])


def _sc_select(x, y, z, seg_s, seg_e):
    mesh = plsc.VectorSubcoreMesh(core_axis_name="c", subcore_axis_name="s")
    run = functools.partial(
        pl.kernel,
        mesh=mesh,
        compiler_params=pltpu.CompilerParams(needs_layout_passes=False),
        out_type=[
            jax.ShapeDtypeStruct((_NPAD * _K2,), jnp.float32),
            jax.ShapeDtypeStruct((_NPAD * _K2,), jnp.int32),
        ],
        scratch_types=[
            pltpu.VMEM((_NPAD,), jnp.float32),
            pltpu.VMEM((_NPAD,), jnp.float32),
            pltpu.VMEM((_NPAD,), jnp.float32),
            pltpu.VMEM((_QPW,), jnp.int32),
            pltpu.VMEM((_QPW,), jnp.int32),
            pltpu.VMEM((_QPW * _K2,), jnp.float32),
            pltpu.VMEM((_QPW * _K2,), jnp.int32),
        ],
    )(_sc_body)
    return run(x, y, z, seg_s, seg_e)


_QT2 = 512


def _tc_body(keys_ref, vals_ref, outi_ref, outw_ref):
    t = pl.program_id(0)
    kiota = lax.broadcasted_iota(jnp.int32, (_QT2, _K), 1)
    qi = (t * _QT2 + lax.broadcasted_iota(jnp.int32, (_QT2, 1), 0))
    keys = keys_ref[...]
    vals = vals_ref[...]
    m_prev = jnp.full((_QT2, 1), -_INF, jnp.float32)
    i_prev = jnp.full((_QT2, 1), -1, jnp.int32)
    acc_i = jnp.broadcast_to(qi, (_QT2, _K))
    acc_w = jnp.zeros((_QT2, _K), jnp.float32)
    for it in range(_K):
        succ = (keys > m_prev) | ((keys == m_prev) & (vals > i_prev))
        cand = jnp.where(succ, keys, _INF)
        cm = jnp.min(cand, axis=1, keepdims=True)
        ci = jnp.min(jnp.where(cand == cm, vals, _BIG), axis=1, keepdims=True)
        valid = cm < _INF
        sel_i = jnp.where(valid, ci, qi)
        sel_w = jnp.where(valid, jnp.sqrt(cm), 0.0)
        hit = kiota == it
        acc_i = jnp.where(hit, jnp.broadcast_to(sel_i, (_QT2, _K)), acc_i)
        acc_w = jnp.where(hit, jnp.broadcast_to(sel_w, (_QT2, _K)), acc_w)
        m_prev, i_prev = cm, ci
    outi_ref[...] = acc_i
    outw_ref[...] = acc_w


def _tc_fixup(keys, vals):
    grid = (_NPAD // _QT2,)
    return pl.pallas_call(
        _tc_body,
        grid=grid,
        in_specs=[
            pl.BlockSpec((_QT2, _K2), lambda t: (t, 0)),
            pl.BlockSpec((_QT2, _K2), lambda t: (t, 0)),
        ],
        out_specs=[
            pl.BlockSpec((_QT2, _K), lambda t: (t, 0)),
            pl.BlockSpec((_QT2, _K), lambda t: (t, 0)),
        ],
        out_shape=[
            jax.ShapeDtypeStruct((_NPAD, _K), jnp.int32),
            jax.ShapeDtypeStruct((_NPAD, _K), jnp.float32),
        ],
    )(keys, vals)


def kernel(pos, batch):
    n = pos.shape[0]
    starts = jnp.searchsorted(
        batch, jnp.arange(17, dtype=jnp.int32), side="left").astype(jnp.int32)
    seg_s = jnp.pad(starts[batch], (0, _NPAD - n))
    seg_e = jnp.pad(starts[batch + 1], (0, _NPAD - n))
    x = jnp.pad(pos[:, 0], (0, _NPAD - n))
    y = jnp.pad(pos[:, 1], (0, _NPAD - n))
    z = jnp.pad(pos[:, 2], (0, _NPAD - n))

    okeys, ovals = _sc_select(x, y, z, seg_s, seg_e)
    out_i, out_w = _tc_fixup(okeys.reshape(_NPAD, _K2),
                             ovals.reshape(_NPAD, _K2))

    row = out_i[:n].reshape(-1)
    col = jnp.broadcast_to(
        jnp.arange(n, dtype=jnp.int32)[:, None], (n, _K)).reshape(-1)
    edge_index = jnp.stack([row, col], axis=0)
    edge_weight = out_w[:n].reshape(-1)
    return edge_index, edge_weight
